# Initial kernel scaffold; baseline (speedup 1.0000x reference)
import jax
import jax.numpy as jnp
from jax.experimental import pallas as pl


def kernel(x, edge_index, W1l, W1r, b1, W2l, W2r, b2):
    raise NotImplementedError("write your pallas kernel here")


# speedup vs baseline: 6.3442x; 6.3442x over previous
import functools

import jax
import jax.numpy as jnp
from jax import lax
from jax.experimental import pallas as pl
from jax.experimental.pallas import tpu as pltpu
from jax.experimental.pallas import tpu_sc as plsc

N, E, D, H, C = 10000, 160000, 256, 16, 40

NC, NS = 2, 16
NW = NC * NS
K = 128
NP = 10240
RPS = NP // NS
EPAD = 163840
CH = EPAD // (NW * K)

_mesh = plsc.VectorSubcoreMesh(core_axis_name="c", subcore_axis_name="s")


def _make_sc_agg(width, with_cnt):
    out_types = [jax.ShapeDtypeStruct((NC, NP, width), jnp.float32)]
    scratch = [
        pltpu.VMEM((K,), jnp.int32),
        pltpu.VMEM((K,), jnp.int32),
        pltpu.VMEM((K, width), jnp.float32),
        pltpu.VMEM_SHARED((NP, width), jnp.float32),
        pltpu.SemaphoreType.DMA,
    ]
    if with_cnt:
        out_types.append(jax.ShapeDtypeStruct((NC, NP, 16), jnp.float32))
        scratch += [
            pltpu.VMEM((K, 16), jnp.float32),
            pltpu.VMEM_SHARED((NP, 16), jnp.float32),
        ]

    def body(table_hbm, src_hbm, dst_hbm, zrow_hbm, *rest):
        if with_cnt:
            (zcnt_hbm, ones_hbm, acc_out, cnt_out,
             src_v, dst_v, rows_v, acc_sh, sem, ones_v, cnt_sh) = rest
        else:
            acc_out, src_v, dst_v, rows_v, acc_sh, sem = rest
        cid = lax.axis_index("c")
        sid = lax.axis_index("s")
        wid = cid * NS + sid
        rbase = sid * RPS
        pltpu.sync_copy(zrow_hbm.at[pl.ds(rbase, RPS)],
                        acc_sh.at[pl.ds(rbase, RPS)])
        if with_cnt:
            pltpu.sync_copy(zcnt_hbm.at[pl.ds(rbase, RPS)],
                            cnt_sh.at[pl.ds(rbase, RPS)])
            pltpu.sync_copy(ones_hbm, ones_v)
        plsc.subcore_barrier()

        def chunk(j, carry):
            base = (wid * CH + j) * K
            pltpu.sync_copy(src_hbm.at[pl.ds(base, K)], src_v)
            pltpu.sync_copy(dst_hbm.at[pl.ds(base, K)], dst_v)
            pltpu.async_copy(table_hbm.at[src_v], rows_v, sem).wait()
            pltpu.sync_copy(rows_v, acc_sh.at[dst_v], add=True)
            if with_cnt:
                pltpu.sync_copy(ones_v, cnt_sh.at[dst_v], add=True)
            return carry

        lax.fori_loop(0, CH, chunk, 0)
        plsc.subcore_barrier()
        pltpu.sync_copy(acc_sh.at[pl.ds(rbase, RPS)],
                        acc_out.at[cid, pl.ds(rbase, RPS)])
        if with_cnt:
            pltpu.sync_copy(cnt_sh.at[pl.ds(rbase, RPS)],
                            cnt_out.at[cid, pl.ds(rbase, RPS)])

    return pl.kernel(body, out_type=tuple(out_types), mesh=_mesh,
                     scratch_types=scratch,
                     compiler_params=pltpu.CompilerParams(
                         use_tc_tiling_on_sc=False))


_sc_agg1 = _make_sc_agg(H, with_cnt=True)
_sc_agg2 = _make_sc_agg(C, with_cnt=False)


_BR = 2048


def _proj1_body(x_ref, wl_ref, wr_ref, y_ref, z_ref):
    xb = x_ref[...]
    y_ref[...] = jnp.dot(xb, wl_ref[...], preferred_element_type=jnp.float32)
    z_ref[...] = jnp.dot(xb, wr_ref[...], preferred_element_type=jnp.float32)


def _mid_body(agg_a, agg_b, cnt_a, cnt_b, z1_ref, b1_ref, wl_ref, wr_ref,
              y2_ref, z2_ref, inv_ref):
    cnt = cnt_a[:, :1] + cnt_b[:, :1]
    inv = 1.0 / jnp.maximum(cnt, 1.0)
    h = jnp.maximum((agg_a[...] + agg_b[...]) * inv + b1_ref[...] + z1_ref[...],
                    0.0)
    y2_ref[...] = jnp.dot(h, wl_ref[...], preferred_element_type=jnp.float32)
    z2_ref[...] = jnp.dot(h, wr_ref[...], preferred_element_type=jnp.float32)
    inv_ref[...] = inv


def _out_body(agg_a, agg_b, inv_ref, z2_ref, b2_ref, o_ref):
    o = (agg_a[...] + agg_b[...]) * inv_ref[...] + b2_ref[...] + z2_ref[...]
    m = jnp.max(o, axis=1, keepdims=True)
    s = jnp.sum(jnp.exp(o - m), axis=1, keepdims=True)
    o_ref[...] = (o - m) - jnp.log(s)


def _row_spec(width):
    return pl.BlockSpec((_BR, width), lambda i: (i, 0))


def _full_spec(r, c):
    return pl.BlockSpec((r, c), lambda i: (0, 0))


_proj1 = pl.pallas_call(
    _proj1_body,
    grid=(NP // _BR,),
    in_specs=[_row_spec(D), _full_spec(D, H), _full_spec(D, H)],
    out_specs=[_row_spec(H), _row_spec(H)],
    out_shape=[jax.ShapeDtypeStruct((NP, H), jnp.float32)] * 2,
)

_mid = pl.pallas_call(
    _mid_body,
    grid=(NP // _BR,),
    in_specs=[_row_spec(H), _row_spec(H), _row_spec(16), _row_spec(16),
              _row_spec(H), _full_spec(1, H), _full_spec(H, C),
              _full_spec(H, C)],
    out_specs=[_row_spec(C), _row_spec(C), _row_spec(1)],
    out_shape=[jax.ShapeDtypeStruct((NP, C), jnp.float32)] * 2
    + [jax.ShapeDtypeStruct((NP, 1), jnp.float32)],
)

_out = pl.pallas_call(
    _out_body,
    grid=(NP // _BR,),
    in_specs=[_row_spec(C), _row_spec(C), _row_spec(1), _row_spec(C),
              _full_spec(1, C)],
    out_specs=_row_spec(C),
    out_shape=jax.ShapeDtypeStruct((NP, C), jnp.float32),
)


def kernel(x, edge_index, W1l, W1r, b1, W2l, W2r, b2):
    xp = jnp.pad(x, ((0, NP - N), (0, 0)))
    src = jnp.concatenate([edge_index[0], jnp.zeros((EPAD - E,), jnp.int32)])
    dst = jnp.concatenate([edge_index[1], jnp.full((EPAD - E,), N, jnp.int32)])
    zrow1 = jnp.zeros((NP, H), jnp.float32)
    zcnt = jnp.zeros((NP, 16), jnp.float32)
    ones = jnp.ones((K, 16), jnp.float32)
    zrow2 = jnp.zeros((NP, C), jnp.float32)

    y1, z1 = _proj1(xp, W1l, W1r)
    agg1, cnt = _sc_agg1(y1, src, dst, zrow1, zcnt, ones)
    y2, z2, inv = _mid(agg1[0], agg1[1], cnt[0], cnt[1], z1,
                       b1.reshape(1, H), W2l, W2r)
    agg2 = _sc_agg2(y2, src, dst, zrow2)
    if isinstance(agg2, (list, tuple)):
        agg2 = agg2[0]
    out = _out(agg2[0], agg2[1], inv, z2, b2.reshape(1, C))
    return out[:N]
